# baseline (speedup 1.0000x reference)
import jax
import jax.numpy as jnp
from jax.experimental import pallas as pl
from jax.experimental.pallas import tpu as pltpu

BATCH = 1024
DIM_CODES = 16
BOOK_SIZE = 1024
EMBEDDING_DIM = 64

BT = 128


def _vq_body(x_ref, cb_ref, dist_ref, idx_ref):
    xb = x_ref[...]
    for c in range(DIM_CODES):
        xc = xb[:, c * EMBEDDING_DIM:(c + 1) * EMBEDDING_DIM]
        cb = cb_ref[c]
        cross = jax.lax.dot_general(
            xc, cb,
            dimension_numbers=(((1,), (1,)), ((), ())),
            preferred_element_type=jnp.float32,
        )
        x_sq = jnp.sum(xc * xc, axis=1, keepdims=True)
        c_sq = jnp.sum(cb * cb, axis=1)
        dist = x_sq - 2.0 * cross + c_sq[None, :]
        dist_ref[:, c * BOOK_SIZE:(c + 1) * BOOK_SIZE] = dist
        m = jnp.min(dist, axis=1, keepdims=True)
        iota = jax.lax.broadcasted_iota(jnp.int32, dist.shape, 1)
        cand = jnp.where(dist == m, iota, BOOK_SIZE)
        idx_ref[:, c] = jnp.min(cand, axis=1)


def kernel(x, codebook):
    batch = x.shape[0]
    dim_codes, book_size, _ = codebook.shape
    grid = (batch // BT,)
    dist2, idx = pl.pallas_call(
        _vq_body,
        grid=grid,
        in_specs=[
            pl.BlockSpec((BT, x.shape[1]), lambda i: (i, 0)),
            pl.BlockSpec(codebook.shape, lambda i: (0, 0, 0)),
        ],
        out_specs=[
            pl.BlockSpec((BT, dim_codes * book_size), lambda i: (i, 0)),
            pl.BlockSpec((BT, dim_codes), lambda i: (i, 0)),
        ],
        out_shape=[
            jax.ShapeDtypeStruct((batch, dim_codes * book_size), jnp.float32),
            jax.ShapeDtypeStruct((batch, dim_codes), jnp.int32),
        ],
    )(x, codebook)
    dist = dist2.reshape(batch, dim_codes, book_size)
    idx_reshaped = idx.astype(jnp.int64)[..., None]
    return (dist, idx_reshaped)


# speedup vs baseline: 1.4031x; 1.4031x over previous
import jax
import jax.numpy as jnp
from jax.experimental import pallas as pl
from jax.experimental.pallas import tpu as pltpu

BATCH = 1024
DIM_CODES = 16
BOOK_SIZE = 1024
EMBEDDING_DIM = 64

BT = 128


def _vq_body(x_ref, cbt_ref, dist_ref, idx_ref, c_sq_ref):
    C, E, K = cbt_ref.shape

    @pl.when(pl.program_id(0) == 0)
    def _():
        cbt = cbt_ref[...]
        c_sq_ref[...] = jnp.sum(cbt * cbt, axis=1)

    xb = x_ref[...]
    xm2 = xb * (-2.0)
    for c in range(C):
        xc = xb[:, c * E:(c + 1) * E]
        xc2 = xm2[:, c * E:(c + 1) * E]
        cross2 = jax.lax.dot_general(
            xc2, cbt_ref[c],
            dimension_numbers=(((1,), (0,)), ((), ())),
            preferred_element_type=jnp.float32,
        )
        x_sq = jnp.sum(xc * xc, axis=1, keepdims=True)
        dist = (x_sq + cross2) + c_sq_ref[c][None, :]
        dist_ref[:, c * K:(c + 1) * K] = dist
        m = jnp.min(dist, axis=1, keepdims=True)
        iota = jax.lax.broadcasted_iota(jnp.int32, dist.shape, 1)
        cand = jnp.where(dist == m, iota, K)
        idx_ref[:, c] = jnp.min(cand, axis=1)


def kernel(x, codebook):
    batch = x.shape[0]
    dim_codes, book_size, emb = codebook.shape
    cbt = jnp.swapaxes(codebook, 1, 2)
    grid = (batch // BT,)
    dist2, idx = pl.pallas_call(
        _vq_body,
        grid=grid,
        in_specs=[
            pl.BlockSpec((BT, x.shape[1]), lambda i: (i, 0)),
            pl.BlockSpec(cbt.shape, lambda i: (0, 0, 0)),
        ],
        out_specs=[
            pl.BlockSpec((BT, dim_codes * book_size), lambda i: (i, 0)),
            pl.BlockSpec((BT, dim_codes), lambda i: (i, 0)),
        ],
        out_shape=[
            jax.ShapeDtypeStruct((batch, dim_codes * book_size), jnp.float32),
            jax.ShapeDtypeStruct((batch, dim_codes), jnp.int32),
        ],
        scratch_shapes=[pltpu.VMEM((dim_codes, book_size), jnp.float32)],
        compiler_params=pltpu.CompilerParams(
            dimension_semantics=("arbitrary",),
        ),
    )(x, cbt)
    dist = dist2.reshape(batch, dim_codes, book_size)
    idx_reshaped = idx.astype(jnp.int64)[..., None]
    return (dist, idx_reshaped)
